# Initial kernel scaffold; baseline (speedup 1.0000x reference)
import jax
import jax.numpy as jnp
from jax.experimental import pallas as pl


def kernel(structure_tokens, table, W1, b1, W2, b2):
    raise NotImplementedError("write your pallas kernel here")


# speedup vs baseline: 12.9586x; 12.9586x over previous
import functools

import jax
import jax.numpy as jnp
from jax import lax
from jax.experimental import pallas as pl
from jax.experimental.pallas import tpu as pltpu
from jax.experimental.pallas import tpu_sc as plsc

_E = 64
_H = 256
_B = 16384
_L = 200
_NC = 2
_NS = 16
_NW = _NC * _NS
_BPW = _B // _NW
_CB = 32
_NBLK = _BPW // _CB
_L0 = 128
_L1 = _L - _L0
_LANES = 16


def _pool_body(tokens_hbm, table_hbm, pooled_hbm, idx_v, g0_v, g1_v, out_v,
               sem_i, sem_g):
    c = lax.axis_index("c")
    s = lax.axis_index("s")
    wid = s * _NC + c
    base = wid * _BPW
    inv_l = jnp.float32(1.0 / _L)

    def do_block(blk, islot):
        def do_row(r, _):
            cp0 = pltpu.async_copy(
                table_hbm.at[idx_v.at[islot, r, pl.ds(0, _L0)]], g0_v, sem_g)
            cp1 = pltpu.async_copy(
                table_hbm.at[idx_v.at[islot, r, pl.ds(_L0, _L1)]], g1_v, sem_g)
            cp0.wait()
            cp1.wait()

            def red0(i, acc):
                return tuple(
                    acc[j] + g0_v[i, pl.ds(j * _LANES, _LANES)]
                    for j in range(4))

            z = jnp.zeros((_LANES,), jnp.float32)
            acc = lax.fori_loop(0, _L0, red0, (z, z, z, z))

            def red1(i, acc):
                return tuple(
                    acc[j] + g1_v[i, pl.ds(j * _LANES, _LANES)]
                    for j in range(4))

            acc = lax.fori_loop(0, _L1, red1, acc)
            row = blk * _CB + r
            for j in range(4):
                out_v[row, pl.ds(j * _LANES, _LANES)] = acc[j] * inv_l
            return 0

        lax.fori_loop(0, _CB, do_row, 0)

    pltpu.sync_copy(tokens_hbm.at[pl.ds(base, _CB)], idx_v.at[0])
    for blk in range(_NBLK):
        islot = blk % 2
        if blk + 1 < _NBLK:
            nxt = pltpu.async_copy(
                tokens_hbm.at[pl.ds(base + (blk + 1) * _CB, _CB)],
                idx_v.at[1 - islot], sem_i)
        do_block(blk, islot)
        if blk + 1 < _NBLK:
            nxt.wait()
    pltpu.sync_copy(out_v, pooled_hbm.at[pl.ds(base, _BPW)])


@jax.jit
def _pool_sc(tokens, table):
    mesh = plsc.VectorSubcoreMesh(core_axis_name="c", subcore_axis_name="s")
    f = pl.kernel(
        _pool_body,
        out_type=jax.ShapeDtypeStruct((_B, _E), jnp.float32),
        mesh=mesh,
        scratch_types=[
            pltpu.VMEM((2, _CB, _L), jnp.int32),
            pltpu.VMEM((_L0, _E), jnp.float32),
            pltpu.VMEM((_L1, _E), jnp.float32),
            pltpu.VMEM((_BPW, _E), jnp.float32),
            pltpu.SemaphoreType.DMA,
            pltpu.SemaphoreType.DMA,
        ],
        compiler_params=pltpu.CompilerParams(use_tc_tiling_on_sc=False),
    )
    return f(tokens, table)


_BB = 512


def _mlp_body(p_ref, w1_ref, b1_ref, w2_ref, b2_ref, o_ref):
    p = p_ref[...]
    h = lax.dot_general(p, w1_ref[...],
                        (((1,), (1,)), ((), ())),
                        preferred_element_type=jnp.float32)
    h = jnp.maximum(h + b1_ref[...], 0.0)
    o = jnp.sum(h * w2_ref[...], axis=1) + b2_ref[0]
    o_ref[...] = jax.nn.sigmoid(o)


@jax.jit
def _mlp_tc(pooled, W1, b1, W2, b2):
    grid = _B // _BB
    return pl.pallas_call(
        _mlp_body,
        grid=(grid,),
        in_specs=[
            pl.BlockSpec((_BB, _E), lambda i: (i, 0)),
            pl.BlockSpec((_H, _E), lambda i: (0, 0)),
            pl.BlockSpec((_H,), lambda i: (0,)),
            pl.BlockSpec((1, _H), lambda i: (0, 0)),
            pl.BlockSpec((1,), lambda i: (0,)),
        ],
        out_specs=pl.BlockSpec((_BB,), lambda i: (i,)),
        out_shape=jax.ShapeDtypeStruct((_B,), jnp.float32),
    )(pooled, W1, b1, W2, b2)


def kernel(structure_tokens, table, W1, b1, W2, b2):
    tokens = structure_tokens.astype(jnp.int32)
    pooled = _pool_sc(tokens, table)
    return _mlp_tc(pooled, W1, b1, W2, b2)


# speedup vs baseline: 24.2195x; 1.8690x over previous
import functools

import jax
import jax.numpy as jnp
from jax import lax
from jax.experimental import pallas as pl
from jax.experimental.pallas import tpu as pltpu
from jax.experimental.pallas import tpu_sc as plsc

_E = 64
_H = 256
_B = 16384
_L = 200
_NC = 2
_NS = 16
_NW = _NC * _NS
_BPW = _B // _NW
_CB = 64
_NBLK = _BPW // _CB
_L0 = 128
_L1 = _L - _L0
_LANES = 16


def _pool_body(tokens_hbm, table_hbm, pooled_hbm, idx_v,
               g0a, g1a, g0b, g1b, out_v, sem_i, sem_a, sem_b):
    c = lax.axis_index("c")
    s = lax.axis_index("s")
    wid = s * _NC + c
    base = wid * _BPW
    inv_l = jnp.float32(1.0 / _L)
    bufs = ((g0a, g1a, sem_a), (g0b, g1b, sem_b))

    def fire_idx(blk, islot):
        pltpu.async_copy(tokens_hbm.at[pl.ds(base + blk * _CB, _CB)],
                         idx_v.at[islot], sem_i)

    def wait_idx():
        pltpu.make_async_copy(tokens_hbm.at[pl.ds(base, _CB)],
                              idx_v.at[0], sem_i).wait()

    def fire_gath(islot, rib, gs):
        g0, g1, sg = bufs[gs]
        pltpu.async_copy(table_hbm.at[idx_v.at[islot, rib, pl.ds(0, _L0)]],
                         g0, sg)
        pltpu.async_copy(table_hbm.at[idx_v.at[islot, rib, pl.ds(_L0, _L1)]],
                         g1, sg)

    def wait_gath(gs):
        g0, g1, sg = bufs[gs]
        pltpu.make_async_copy(
            table_hbm.at[idx_v.at[0, 0, pl.ds(0, _L0)]], g0, sg).wait()
        pltpu.make_async_copy(
            table_hbm.at[idx_v.at[0, 0, pl.ds(_L0, _L1)]], g1, sg).wait()

    def reduce_row(gs, row):
        g0, g1, _ = bufs[gs]

        def red0(i, acc):
            return tuple(acc[j] + g0[i, pl.ds(j * _LANES, _LANES)]
                         for j in range(4))

        z = jnp.zeros((_LANES,), jnp.float32)
        acc = lax.fori_loop(0, _L0, red0, (z, z, z, z), unroll=8)

        def red1(i, acc):
            return tuple(acc[j] + g1[i, pl.ds(j * _LANES, _LANES)]
                         for j in range(4))

        acc = lax.fori_loop(0, _L1, red1, acc, unroll=8)
        for j in range(4):
            out_v[row, pl.ds(j * _LANES, _LANES)] = acc[j] * inv_l

    pltpu.sync_copy(tokens_hbm.at[pl.ds(base, _CB)], idx_v.at[0])
    fire_gath(0, 0, 0)
    for blk in range(_NBLK):
        islot = blk % 2
        if blk + 1 < _NBLK:
            fire_idx(blk + 1, 1 - islot)
        rowbase = blk * _CB

        @pl.loop(0, _CB // 2 - 1)
        def _(p):
            r0 = 2 * p
            fire_gath(islot, r0 + 1, 1)
            wait_gath(0)
            reduce_row(0, rowbase + r0)
            fire_gath(islot, r0 + 2, 0)
            wait_gath(1)
            reduce_row(1, rowbase + r0 + 1)

        fire_gath(islot, _CB - 1, 1)
        wait_gath(0)
        reduce_row(0, rowbase + _CB - 2)
        if blk + 1 < _NBLK:
            wait_idx()
            fire_gath(1 - islot, 0, 0)
        wait_gath(1)
        reduce_row(1, rowbase + _CB - 1)
    pltpu.sync_copy(out_v, pooled_hbm.at[pl.ds(base, _BPW)])


@jax.jit
def _pool_sc(tokens, table):
    mesh = plsc.VectorSubcoreMesh(core_axis_name="c", subcore_axis_name="s")
    f = pl.kernel(
        _pool_body,
        out_type=jax.ShapeDtypeStruct((_B, _E), jnp.float32),
        mesh=mesh,
        scratch_types=[
            pltpu.VMEM((2, _CB, _L), jnp.int32),
            pltpu.VMEM((_L0, _E), jnp.float32),
            pltpu.VMEM((_L1, _E), jnp.float32),
            pltpu.VMEM((_L0, _E), jnp.float32),
            pltpu.VMEM((_L1, _E), jnp.float32),
            pltpu.VMEM((_BPW, _E), jnp.float32),
            pltpu.SemaphoreType.DMA,
            pltpu.SemaphoreType.DMA,
            pltpu.SemaphoreType.DMA,
        ],
        compiler_params=pltpu.CompilerParams(use_tc_tiling_on_sc=False),
    )
    return f(tokens, table)


_BB = 512


def _mlp_body(p_ref, w1_ref, b1_ref, w2_ref, b2_ref, o_ref):
    p = p_ref[...]
    h = lax.dot_general(p, w1_ref[...],
                        (((1,), (1,)), ((), ())),
                        preferred_element_type=jnp.float32)
    h = jnp.maximum(h + b1_ref[...], 0.0)
    o = jnp.sum(h * w2_ref[...], axis=1) + b2_ref[0]
    o_ref[...] = jax.nn.sigmoid(o)


@jax.jit
def _mlp_tc(pooled, W1, b1, W2, b2):
    grid = _B // _BB
    return pl.pallas_call(
        _mlp_body,
        grid=(grid,),
        in_specs=[
            pl.BlockSpec((_BB, _E), lambda i: (i, 0)),
            pl.BlockSpec((_H, _E), lambda i: (0, 0)),
            pl.BlockSpec((_H,), lambda i: (0,)),
            pl.BlockSpec((1, _H), lambda i: (0, 0)),
            pl.BlockSpec((1,), lambda i: (0,)),
        ],
        out_specs=pl.BlockSpec((_BB,), lambda i: (i,)),
        out_shape=jax.ShapeDtypeStruct((_B,), jnp.float32),
    )(pooled, W1, b1, W2, b2)


def kernel(structure_tokens, table, W1, b1, W2, b2):
    tokens = structure_tokens.astype(jnp.int32)
    pooled = _pool_sc(tokens, table)
    return _mlp_tc(pooled, W1, b1, W2, b2)


# speedup vs baseline: 33.6143x; 1.3879x over previous
import functools

import jax
import jax.numpy as jnp
from jax import lax
from jax.experimental import pallas as pl
from jax.experimental.pallas import tpu as pltpu
from jax.experimental.pallas import tpu_sc as plsc

_E = 64
_H = 256
_B = 16384
_L = 200
_NC = 2
_NS = 16
_NW = _NC * _NS
_BPW = _B // _NW
_CB = 64
_NBLK = _BPW // _CB
_L0 = 128
_L1 = _L - _L0
_LANES = 16


def _pool_body(tokens_hbm, table_hbm, pooled_hbm, idx_v,
               g0a, g1a, g0b, g1b, g0c, g1c, g0d, g1d, out_v,
               sem_i, sem_a, sem_b, sem_c, sem_d):
    c = lax.axis_index("c")
    s = lax.axis_index("s")
    wid = s * _NC + c
    base = wid * _BPW
    inv_l = jnp.float32(1.0 / _L)
    bufs = ((g0a, g1a, sem_a), (g0b, g1b, sem_b),
            (g0c, g1c, sem_c), (g0d, g1d, sem_d))

    def fire_idx(blk, islot):
        pltpu.async_copy(tokens_hbm.at[pl.ds(base + blk * _CB, _CB)],
                         idx_v.at[islot], sem_i)

    def wait_idx():
        pltpu.make_async_copy(tokens_hbm.at[pl.ds(base, _CB)],
                              idx_v.at[0], sem_i).wait()

    def fire_gath(islot, rib, gs):
        g0, g1, sg = bufs[gs]
        pltpu.async_copy(table_hbm.at[idx_v.at[islot, rib, pl.ds(0, _L0)]],
                         g0, sg)
        pltpu.async_copy(table_hbm.at[idx_v.at[islot, rib, pl.ds(_L0, _L1)]],
                         g1, sg)

    def wait_gath(gs):
        g0, g1, sg = bufs[gs]
        pltpu.make_async_copy(
            table_hbm.at[idx_v.at[0, 0, pl.ds(0, _L0)]], g0, sg).wait()
        pltpu.make_async_copy(
            table_hbm.at[idx_v.at[0, 0, pl.ds(_L0, _L1)]], g1, sg).wait()

    def reduce_row(gs, row):
        g0, g1, _ = bufs[gs]

        def red0(i, acc):
            return tuple(acc[j] + g0[i, pl.ds(j * _LANES, _LANES)]
                         for j in range(4))

        z = jnp.zeros((_LANES,), jnp.float32)
        acc = lax.fori_loop(0, _L0, red0, (z, z, z, z), unroll=8)

        def red1(i, acc):
            return tuple(acc[j] + g1[i, pl.ds(j * _LANES, _LANES)]
                         for j in range(4))

        acc = lax.fori_loop(0, _L1, red1, acc, unroll=8)
        for j in range(4):
            out_v[row, pl.ds(j * _LANES, _LANES)] = acc[j] * inv_l

    pltpu.sync_copy(tokens_hbm.at[pl.ds(base, _CB)], idx_v.at[0])
    for k in range(3):
        fire_gath(0, k, k)

    @pl.loop(0, _NBLK, step=2)
    def _(blk0):
        for di in range(2):
            blk = blk0 + di
            islot = di
            nislot = 1 - di
            rowbase = blk * _CB
            has_next = blk + 1 < _NBLK

            @pl.when(has_next)
            def _():
                fire_idx(blk + 1, nislot)

            @pl.loop(0, (_CB - 4) // 4)
            def _(p):
                r = 4 * p
                for k in range(4):
                    fire_gath(islot, r + k + 3, (k + 3) % 4)
                    wait_gath(k)
                    reduce_row(k, rowbase + r + k)

            rt = _CB - 4
            fire_gath(islot, _CB - 1, 3)
            wait_gath(0)
            reduce_row(0, rowbase + rt)

            @pl.when(has_next)
            def _():
                wait_idx()
                fire_gath(nislot, 0, 0)

            wait_gath(1)
            reduce_row(1, rowbase + rt + 1)

            @pl.when(has_next)
            def _():
                fire_gath(nislot, 1, 1)

            wait_gath(2)
            reduce_row(2, rowbase + rt + 2)

            @pl.when(has_next)
            def _():
                fire_gath(nislot, 2, 2)

            wait_gath(3)
            reduce_row(3, rowbase + rt + 3)

    pltpu.sync_copy(out_v, pooled_hbm.at[pl.ds(base, _BPW)])


@jax.jit
def _pool_sc(tokens, table):
    mesh = plsc.VectorSubcoreMesh(core_axis_name="c", subcore_axis_name="s")
    f = pl.kernel(
        _pool_body,
        out_type=jax.ShapeDtypeStruct((_B, _E), jnp.float32),
        mesh=mesh,
        scratch_types=(
            [pltpu.VMEM((2, _CB, _L), jnp.int32)]
            + [pltpu.VMEM((n, _E), jnp.float32)
               for _ in range(4) for n in (_L0, _L1)]
            + [pltpu.VMEM((_BPW, _E), jnp.float32)]
            + [pltpu.SemaphoreType.DMA] * 5
        ),
        compiler_params=pltpu.CompilerParams(use_tc_tiling_on_sc=False),
    )
    return f(tokens, table)


_BB = 512


def _mlp_body(p_ref, w1_ref, b1_ref, w2_ref, b2_ref, o_ref):
    p = p_ref[...]
    h = lax.dot_general(p, w1_ref[...],
                        (((1,), (1,)), ((), ())),
                        preferred_element_type=jnp.float32)
    h = jnp.maximum(h + b1_ref[...], 0.0)
    o = jnp.sum(h * w2_ref[...], axis=1) + b2_ref[0]
    o_ref[...] = jax.nn.sigmoid(o)


@jax.jit
def _mlp_tc(pooled, W1, b1, W2, b2):
    grid = _B // _BB
    return pl.pallas_call(
        _mlp_body,
        grid=(grid,),
        in_specs=[
            pl.BlockSpec((_BB, _E), lambda i: (i, 0)),
            pl.BlockSpec((_H, _E), lambda i: (0, 0)),
            pl.BlockSpec((_H,), lambda i: (0,)),
            pl.BlockSpec((1, _H), lambda i: (0, 0)),
            pl.BlockSpec((1,), lambda i: (0,)),
        ],
        out_specs=pl.BlockSpec((_BB,), lambda i: (i,)),
        out_shape=jax.ShapeDtypeStruct((_B,), jnp.float32),
    )(pooled, W1, b1, W2, b2)


def kernel(structure_tokens, table, W1, b1, W2, b2):
    tokens = structure_tokens.astype(jnp.int32)
    pooled = _pool_sc(tokens, table)
    return _mlp_tc(pooled, W1, b1, W2, b2)
